# baseline (speedup 1.0000x reference)
import functools

import jax
import jax.numpy as jnp
from jax import lax
from jax.experimental import pallas as pl
from jax.experimental.pallas import tpu as pltpu
from jax.experimental.pallas import tpu_sc as plsc

VOCAB = 100000
EMBD = 64
B = 1024

NC = 2
NS = 16
NW = NC * NS
BPW = B // NW

TILE_V = 2048


def _gather_body(table_hbm, idx_hbm, out_hbm, idx_v, rows_v, sem):
    wid = lax.axis_index("s") * NC + lax.axis_index("c")
    base = wid * BPW
    pltpu.sync_copy(idx_hbm.at[pl.ds(base, BPW)], idx_v)
    pltpu.async_copy(table_hbm.at[idx_v], rows_v, sem).wait()
    pltpu.sync_copy(rows_v, out_hbm.at[pl.ds(base, BPW)])


def _sc_gather(table, idx):
    mesh = plsc.VectorSubcoreMesh(core_axis_name="c", subcore_axis_name="s")
    return pl.kernel(
        _gather_body,
        mesh=mesh,
        out_type=jax.ShapeDtypeStruct((B, EMBD), jnp.float32),
        scratch_types=[
            pltpu.VMEM((BPW,), jnp.int32),
            pltpu.VMEM((BPW, EMBD), jnp.float32),
            pltpu.SemaphoreType.DMA,
        ],
        compiler_params=pltpu.CompilerParams(use_tc_tiling_on_sc=False),
    )(table, idx)


def _mm_body(emb_ref, w_ref, b_ref, out_ref):
    out_ref[...] = (
        jnp.dot(emb_ref[...], w_ref[...], preferred_element_type=jnp.float32)
        + b_ref[...]
    )


def _tc_matmul(tok_emb, W, b2):
    n_tiles = pl.cdiv(VOCAB, TILE_V)
    return pl.pallas_call(
        _mm_body,
        grid=(n_tiles,),
        in_specs=[
            pl.BlockSpec((B, EMBD), lambda i: (0, 0)),
            pl.BlockSpec((EMBD, TILE_V), lambda i: (0, i)),
            pl.BlockSpec((1, TILE_V), lambda i: (0, i)),
        ],
        out_specs=pl.BlockSpec((B, TILE_V), lambda i: (0, i)),
        out_shape=jax.ShapeDtypeStruct((B, VOCAB), jnp.float32),
        compiler_params=pltpu.CompilerParams(
            dimension_semantics=("arbitrary",),
        ),
    )(tok_emb, W, b2)


@jax.jit
def kernel(idx, table, W, b):
    tok_emb = _sc_gather(table, idx.astype(jnp.int32))
    return _tc_matmul(tok_emb, W, b.reshape(1, VOCAB))


# speedup vs baseline: 1.0524x; 1.0524x over previous
import functools

import jax
import jax.numpy as jnp
from jax import lax
from jax.experimental import pallas as pl
from jax.experimental.pallas import tpu as pltpu
from jax.experimental.pallas import tpu_sc as plsc

VOCAB = 100000
EMBD = 64
B = 1024

NC = 2
NS = 16
NW = NC * NS
BPW = B // NW

TILE_V = 2048


def _gather_body(table_hbm, idx_hbm, out_hbm, idx_v, rows_v, sem):
    wid = lax.axis_index("s") * NC + lax.axis_index("c")
    base = wid * BPW
    pltpu.sync_copy(idx_hbm.at[pl.ds(base, BPW)], idx_v)
    pltpu.async_copy(table_hbm.at[idx_v], rows_v, sem).wait()
    pltpu.sync_copy(rows_v, out_hbm.at[pl.ds(base, BPW)])


def _sc_gather(table, idx):
    mesh = plsc.VectorSubcoreMesh(core_axis_name="c", subcore_axis_name="s")
    return pl.kernel(
        _gather_body,
        mesh=mesh,
        out_type=jax.ShapeDtypeStruct((B, EMBD), jnp.float32),
        scratch_types=[
            pltpu.VMEM((BPW,), jnp.int32),
            pltpu.VMEM((BPW, EMBD), jnp.float32),
            pltpu.SemaphoreType.DMA,
        ],
        compiler_params=pltpu.CompilerParams(use_tc_tiling_on_sc=False),
    )(table, idx)


def _mm_body(emb_ref, w_ref, b_ref, out_ref):
    out_ref[...] = (
        jnp.dot(emb_ref[...], w_ref[...], preferred_element_type=jnp.float32)
        + b_ref[...]
    )


def _tc_matmul(tok_emb, W, b2):
    n_tiles = pl.cdiv(VOCAB, TILE_V)
    return pl.pallas_call(
        _mm_body,
        grid=(n_tiles,),
        in_specs=[
            pl.BlockSpec((B, EMBD), lambda i: (0, 0)),
            pl.BlockSpec((EMBD, TILE_V), lambda i: (0, i)),
            pl.BlockSpec((1, TILE_V), lambda i: (0, i)),
        ],
        out_specs=pl.BlockSpec((B, TILE_V), lambda i: (0, i)),
        out_shape=jax.ShapeDtypeStruct((B, VOCAB), jnp.float32),
        compiler_params=pltpu.CompilerParams(
            dimension_semantics=("arbitrary",),
        ),
    )(tok_emb, W, b2)


@jax.jit
def kernel(idx, table, W, b):
    tok_emb = jnp.take(table, idx, axis=0)
    return _tc_matmul(tok_emb, W, b.reshape(1, VOCAB))
